# Initial kernel scaffold; baseline (speedup 1.0000x reference)
import jax
import jax.numpy as jnp
from jax.experimental import pallas as pl


def kernel(x_0, x_0_batch, num_cells_0, W1, b1, g1, bt1, W2, b2, g2, bt2, W3, b3):
    raise NotImplementedError("write your pallas kernel here")


# speedup vs baseline: 4.5960x; 4.5960x over previous
import functools

import jax
import jax.numpy as jnp
from jax import lax
from jax.experimental import pallas as pl
from jax.experimental.pallas import tpu as pltpu
from jax.experimental.pallas import tpu_sc as plsc

N = 100000
EMB = 128
BATCH = 1024
OUT = 1

NC = 2
NS = 16
NW = NC * NS
CHUNK = 128
ROUNDS = 25
PAD_ROWS = NW * ROUNDS * CHUNK
PARTIAL = N % CHUNK
ACC_ROWS = 1152
ZROWS = ACC_ROWS // NS

_BN_INV = 0.9999950000374997


@functools.cache
def _make_sc_segment_sum():
    mesh = plsc.VectorSubcoreMesh(
        core_axis_name="c", subcore_axis_name="s",
        num_cores=NC, num_subcores=NS)
    return pl.kernel(
        _sc_segment_sum_body,
        out_type=jax.ShapeDtypeStruct((NC, BATCH, EMB), jnp.float32),
        mesh=mesh,
        scratch_types=[
            pltpu.VMEM((ROUNDS, CHUNK), jnp.int32),
            pltpu.VMEM((CHUNK, EMB), jnp.float32),
            pltpu.VMEM((ZROWS, EMB), jnp.float32),
            pltpu.VMEM_SHARED((ACC_ROWS, EMB), jnp.float32),
        ],
    )


def _sc_segment_sum_body(x_hbm, idx_hbm, zeros_hbm, out_hbm,
                         idx_v, buf_v, bounce_v, acc_sh):
    c = lax.axis_index("c")
    s = lax.axis_index("s")
    w = c * NS + s

    pltpu.sync_copy(zeros_hbm, bounce_v)
    pltpu.sync_copy(bounce_v, acc_sh.at[pl.ds(s * ZROWS, ZROWS)])
    plsc.subcore_barrier()

    pltpu.sync_copy(idx_hbm.at[w], idx_v)

    for r in range(ROUNDS):
        start = (w * ROUNDS + r) * CHUNK

        @pl.when(start + CHUNK <= N)
        def _full():
            pltpu.sync_copy(x_hbm.at[pl.ds(start, CHUNK)], buf_v)
            pltpu.sync_copy(buf_v, acc_sh.at[idx_v.at[r]], add=True)

        @pl.when(start == N - PARTIAL)
        def _partial():
            pltpu.sync_copy(x_hbm.at[pl.ds(start, PARTIAL)],
                            buf_v.at[pl.ds(0, PARTIAL)])
            pltpu.sync_copy(buf_v, acc_sh.at[idx_v.at[r]], add=True)

    plsc.subcore_barrier()

    wrows = BATCH // NS
    pltpu.sync_copy(acc_sh.at[pl.ds(s * wrows, wrows)],
                    bounce_v.at[pl.ds(0, wrows)])
    pltpu.sync_copy(bounce_v.at[pl.ds(0, wrows)],
                    out_hbm.at[c].at[pl.ds(s * wrows, wrows)])


def _head_body(p_ref, w1, b1, g1, bt1, w2, b2, g2, bt2, w3t, b3, o_ref):
    pooled = p_ref[0] + p_ref[1]
    h = jnp.dot(pooled, w1[...], preferred_element_type=jnp.float32) + b1[...]
    h = jnp.maximum(h * (g1[...] * _BN_INV) + bt1[...], 0.0)
    h = jnp.dot(h, w2[...], preferred_element_type=jnp.float32) + b2[...]
    h = jnp.maximum(h * (g2[...] * _BN_INV) + bt2[...], 0.0)
    o_ref[...] = jnp.sum(h * w3t[...], axis=1, keepdims=True) + b3[...]


def _head(partials, W1, b1, g1, bt1, W2, b2, g2, bt2, W3, b3):
    row = lambda v: v.reshape(1, -1)
    return pl.pallas_call(
        _head_body,
        out_shape=jax.ShapeDtypeStruct((BATCH, OUT), jnp.float32),
    )(partials, W1, row(b1), row(g1), row(bt1),
      W2, row(b2), row(g2), row(bt2),
      W3.reshape(1, 2 * EMB), b3.reshape(1, 1))


def kernel(x_0, x_0_batch, num_cells_0, W1, b1, g1, bt1, W2, b2, g2, bt2, W3, b3):
    idx = jnp.squeeze(x_0_batch).astype(jnp.int32)
    pad = jnp.full((PAD_ROWS - N,), BATCH, jnp.int32)
    idx2 = jnp.concatenate([idx, pad]).reshape(NW, ROUNDS, CHUNK)
    zeros = jnp.zeros((ZROWS, EMB), jnp.float32)
    partials = _make_sc_segment_sum()(x_0, idx2, zeros)
    return _head(partials, W1, b1, g1, bt1, W2, b2, g2, bt2, W3, b3)


# speedup vs baseline: 5.9963x; 1.3047x over previous
import functools

import jax
import jax.numpy as jnp
from jax import lax
from jax.experimental import pallas as pl
from jax.experimental.pallas import tpu as pltpu
from jax.experimental.pallas import tpu_sc as plsc

N = 100000
EMB = 128
BATCH = 1024
OUT = 1

NC = 2
NS = 16
NW = NC * NS
CHUNK = 128
ROUNDS = 25
PAD_ROWS = NW * ROUNDS * CHUNK
PARTIAL = N % CHUNK
ACC_ROWS = 1152
ZROWS = ACC_ROWS // NS

_BN_INV = 0.9999950000374997


@functools.cache
def _make_sc_segment_sum():
    mesh = plsc.VectorSubcoreMesh(
        core_axis_name="c", subcore_axis_name="s",
        num_cores=NC, num_subcores=NS)
    return pl.kernel(
        _sc_segment_sum_body,
        out_type=jax.ShapeDtypeStruct((NC, BATCH, EMB), jnp.float32),
        mesh=mesh,
        scratch_types=[
            pltpu.VMEM((ROUNDS, CHUNK), jnp.int32),
            pltpu.VMEM((CHUNK, EMB), jnp.float32),
            pltpu.VMEM((CHUNK, EMB), jnp.float32),
            pltpu.VMEM((ZROWS, EMB), jnp.float32),
            pltpu.VMEM_SHARED((ACC_ROWS, EMB), jnp.float32),
            pltpu.SemaphoreType.DMA,
            pltpu.SemaphoreType.DMA,
        ],
    )


def _sc_segment_sum_body(x_hbm, idx_hbm, zeros_hbm, out_hbm,
                         idx_v, buf_a, buf_b, bounce_v, acc_sh,
                         sem_a, sem_b):
    c = lax.axis_index("c")
    s = lax.axis_index("s")
    w = c * NS + s

    bufs = (buf_a, buf_b)
    sems = (sem_a, sem_b)

    def row_start(r):
        return (w * ROUNDS + r) * CHUNK

    def full_desc(r):
        return pltpu.make_async_copy(
            x_hbm.at[pl.ds(row_start(r), CHUNK)], bufs[r % 2], sems[r % 2])

    def part_desc(r):
        return pltpu.make_async_copy(
            x_hbm.at[pl.ds(row_start(r), PARTIAL)],
            bufs[r % 2].at[pl.ds(0, PARTIAL)], sems[r % 2])

    def issue(r):
        @pl.when(row_start(r) + CHUNK <= N)
        def _():
            full_desc(r).start()

        @pl.when(row_start(r) == N - PARTIAL)
        def _():
            part_desc(r).start()

    issue(0)

    pltpu.sync_copy(zeros_hbm, bounce_v)
    pltpu.sync_copy(bounce_v, acc_sh.at[pl.ds(s * ZROWS, ZROWS)])

    pltpu.sync_copy(idx_hbm.at[w], idx_v)
    plsc.subcore_barrier()

    for r in range(ROUNDS):
        if r + 1 < ROUNDS:
            issue(r + 1)

        @pl.when(row_start(r) + CHUNK <= N)
        def _full():
            full_desc(r).wait()
            pltpu.sync_copy(bufs[r % 2], acc_sh.at[idx_v.at[r]], add=True)

        @pl.when(row_start(r) == N - PARTIAL)
        def _partial():
            part_desc(r).wait()
            pltpu.sync_copy(bufs[r % 2], acc_sh.at[idx_v.at[r]], add=True)

    plsc.subcore_barrier()

    wrows = BATCH // NS
    pltpu.sync_copy(acc_sh.at[pl.ds(s * wrows, wrows)],
                    bounce_v.at[pl.ds(0, wrows)])
    pltpu.sync_copy(bounce_v.at[pl.ds(0, wrows)],
                    out_hbm.at[c].at[pl.ds(s * wrows, wrows)])


def _head_body(p_ref, w1, b1, g1, bt1, w2, b2, g2, bt2, w3t, b3, o_ref):
    pooled = p_ref[0] + p_ref[1]
    h = jnp.dot(pooled, w1[...], preferred_element_type=jnp.float32) + b1[...]
    h = jnp.maximum(h * (g1[...] * _BN_INV) + bt1[...], 0.0)
    h = jnp.dot(h, w2[...], preferred_element_type=jnp.float32) + b2[...]
    h = jnp.maximum(h * (g2[...] * _BN_INV) + bt2[...], 0.0)
    o_ref[...] = jnp.sum(h * w3t[...], axis=1, keepdims=True) + b3[...]


def _head(partials, W1, b1, g1, bt1, W2, b2, g2, bt2, W3, b3):
    row = lambda v: v.reshape(1, -1)
    return pl.pallas_call(
        _head_body,
        out_shape=jax.ShapeDtypeStruct((BATCH, OUT), jnp.float32),
    )(partials, W1, row(b1), row(g1), row(bt1),
      W2, row(b2), row(g2), row(bt2),
      W3.reshape(1, 2 * EMB), b3.reshape(1, 1))


def kernel(x_0, x_0_batch, num_cells_0, W1, b1, g1, bt1, W2, b2, g2, bt2, W3, b3):
    idx = jnp.squeeze(x_0_batch).astype(jnp.int32)
    pad = jnp.full((PAD_ROWS - N,), BATCH, jnp.int32)
    idx2 = jnp.concatenate([idx, pad]).reshape(NW, ROUNDS, CHUNK)
    zeros = jnp.zeros((ZROWS, EMB), jnp.float32)
    partials = _make_sc_segment_sum()(x_0, idx2, zeros)
    return _head(partials, W1, b1, g1, bt1, W2, b2, g2, bt2, W3, b3)


# speedup vs baseline: 6.3494x; 1.0589x over previous
import functools

import jax
import jax.numpy as jnp
from jax import lax
from jax.experimental import pallas as pl
from jax.experimental.pallas import tpu as pltpu
from jax.experimental.pallas import tpu_sc as plsc

N = 100000
EMB = 128
BATCH = 1024
OUT = 1

NC = 2
NS = 16
NW = NC * NS
CHUNK = 128
NBUF = 4
ROUNDS = 25
PAD_ROWS = NW * ROUNDS * CHUNK
PARTIAL = N % CHUNK
ACC_ROWS = 1152
ZROWS = ACC_ROWS // NS

_BN_INV = 0.9999950000374997


@functools.cache
def _make_sc_segment_sum():
    mesh = plsc.VectorSubcoreMesh(
        core_axis_name="c", subcore_axis_name="s",
        num_cores=NC, num_subcores=NS)
    return pl.kernel(
        _sc_segment_sum_body,
        out_type=jax.ShapeDtypeStruct((NC, BATCH, EMB), jnp.float32),
        mesh=mesh,
        scratch_types=[
            pltpu.VMEM((ROUNDS, CHUNK), jnp.int32),
            [pltpu.VMEM((CHUNK, EMB), jnp.float32) for _ in range(NBUF)],
            pltpu.VMEM((ZROWS, EMB), jnp.float32),
            pltpu.VMEM_SHARED((ACC_ROWS, EMB), jnp.float32),
            [pltpu.SemaphoreType.DMA for _ in range(NBUF)],
            [pltpu.SemaphoreType.DMA for _ in range(NBUF)],
        ],
    )


def _sc_segment_sum_body(x_hbm, idx_hbm, zeros_hbm, out_hbm,
                         idx_v, bufs, bounce_v, acc_sh, lsems, ssems):
    c = lax.axis_index("c")
    s = lax.axis_index("s")
    w = c * NS + s

    def row_start(r):
        return (w * ROUNDS + r) * CHUNK

    def cond_full(r):
        return row_start(r) + CHUNK <= N

    def cond_part(r):
        return row_start(r) == N - PARTIAL

    def load_full(r):
        return pltpu.make_async_copy(
            x_hbm.at[pl.ds(row_start(r), CHUNK)], bufs[r % NBUF],
            lsems[r % NBUF])

    def load_part(r):
        return pltpu.make_async_copy(
            x_hbm.at[pl.ds(row_start(r), PARTIAL)],
            bufs[r % NBUF].at[pl.ds(0, PARTIAL)], lsems[r % NBUF])

    def scat(r):
        return pltpu.make_async_copy(
            bufs[r % NBUF], acc_sh.at[idx_v.at[r]], ssems[r % NBUF])

    def issue_load(r):
        @pl.when(cond_full(r))
        def _():
            load_full(r).start()

        @pl.when(cond_part(r))
        def _():
            load_part(r).start()

    for k in range(NBUF - 1):
        issue_load(k)

    pltpu.sync_copy(zeros_hbm, bounce_v)
    pltpu.sync_copy(bounce_v, acc_sh.at[pl.ds(s * ZROWS, ZROWS)])

    pltpu.sync_copy(idx_hbm.at[w], idx_v)
    plsc.subcore_barrier()

    for r in range(ROUNDS):
        @pl.when(cond_full(r))
        def _full():
            load_full(r).wait()
            scat(r).start(add=True)

        @pl.when(cond_part(r))
        def _partial():
            load_part(r).wait()
            scat(r).start(add=True)

        nxt = r + NBUF - 1
        if nxt < ROUNDS:
            prev = nxt - NBUF
            if prev >= 0:
                @pl.when(cond_full(prev) | cond_part(prev))
                def _drain():
                    scat(prev).wait()
            issue_load(nxt)

    for r in range(max(0, ROUNDS - NBUF), ROUNDS):
        @pl.when(cond_full(r) | cond_part(r))
        def _drain_tail():
            scat(r).wait()

    plsc.subcore_barrier()

    wrows = BATCH // NS
    pltpu.sync_copy(acc_sh.at[pl.ds(s * wrows, wrows)],
                    bounce_v.at[pl.ds(0, wrows)])
    pltpu.sync_copy(bounce_v.at[pl.ds(0, wrows)],
                    out_hbm.at[c].at[pl.ds(s * wrows, wrows)])


def _head_body(p_ref, w1, b1, g1, bt1, w2, b2, g2, bt2, w3t, b3, o_ref):
    pooled = p_ref[0] + p_ref[1]
    h = jnp.dot(pooled, w1[...], preferred_element_type=jnp.float32) + b1[...]
    h = jnp.maximum(h * (g1[...] * _BN_INV) + bt1[...], 0.0)
    h = jnp.dot(h, w2[...], preferred_element_type=jnp.float32) + b2[...]
    h = jnp.maximum(h * (g2[...] * _BN_INV) + bt2[...], 0.0)
    o_ref[...] = jnp.sum(h * w3t[...], axis=1, keepdims=True) + b3[...]


def _head(partials, W1, b1, g1, bt1, W2, b2, g2, bt2, W3, b3):
    row = lambda v: v.reshape(1, -1)
    return pl.pallas_call(
        _head_body,
        out_shape=jax.ShapeDtypeStruct((BATCH, OUT), jnp.float32),
    )(partials, W1, row(b1), row(g1), row(bt1),
      W2, row(b2), row(g2), row(bt2),
      W3.reshape(1, 2 * EMB), b3.reshape(1, 1))


def kernel(x_0, x_0_batch, num_cells_0, W1, b1, g1, bt1, W2, b2, g2, bt2, W3, b3):
    idx = jnp.squeeze(x_0_batch).astype(jnp.int32)
    pad = jnp.full((PAD_ROWS - N,), BATCH, jnp.int32)
    idx2 = jnp.concatenate([idx, pad]).reshape(NW, ROUNDS, CHUNK)
    zeros = jnp.zeros((ZROWS, EMB), jnp.float32)
    partials = _make_sc_segment_sum()(x_0, idx2, zeros)
    return _head(partials, W1, b1, g1, bt1, W2, b2, g2, bt2, W3, b3)


# speedup vs baseline: 6.6020x; 1.0398x over previous
import functools

import jax
import jax.numpy as jnp
from jax import lax
from jax.experimental import pallas as pl
from jax.experimental.pallas import tpu as pltpu
from jax.experimental.pallas import tpu_sc as plsc

N = 100000
EMB = 128
BATCH = 1024
OUT = 1

NC = 2
NS = 16
NW = NC * NS
CHUNK = 128
NBUF = 4
ROUNDS = 25
PAD_ROWS = NW * ROUNDS * CHUNK
PARTIAL = N % CHUNK
ACC_ROWS = 1152
ZROWS = ACC_ROWS // NS

_BN_INV = 0.9999950000374997


@functools.cache
def _make_sc_segment_sum():
    mesh = plsc.VectorSubcoreMesh(
        core_axis_name="c", subcore_axis_name="s",
        num_cores=NC, num_subcores=NS)
    return pl.kernel(
        _sc_segment_sum_body,
        out_type=jax.ShapeDtypeStruct((NC, BATCH, EMB), jnp.float32),
        mesh=mesh,
        scratch_types=[
            pltpu.VMEM((ROUNDS, CHUNK), jnp.int32),
            [pltpu.VMEM((CHUNK, EMB), jnp.float32) for _ in range(NBUF)],
            pltpu.VMEM((ZROWS, EMB), jnp.float32),
            pltpu.VMEM_SHARED((ACC_ROWS, EMB), jnp.float32),
            [pltpu.SemaphoreType.DMA for _ in range(NBUF)],
            [pltpu.SemaphoreType.DMA for _ in range(NBUF)],
            pltpu.SemaphoreType.DMA,
        ],
    )


def _sc_segment_sum_body(x_hbm, idx_hbm, out_hbm,
                         idx_v, bufs, bounce_v, acc_sh, lsems, ssems, isem):
    c = lax.axis_index("c")
    s = lax.axis_index("s")
    w = c * NS + s

    def row_start(r):
        return (w * ROUNDS + r) * CHUNK

    def cond_full(r):
        return row_start(r) + CHUNK <= N

    def cond_part(r):
        return row_start(r) == N - PARTIAL

    def load_full(r):
        return pltpu.make_async_copy(
            x_hbm.at[pl.ds(row_start(r), CHUNK)], bufs[r % NBUF],
            lsems[r % NBUF])

    def load_part(r):
        return pltpu.make_async_copy(
            x_hbm.at[pl.ds(row_start(r), PARTIAL)],
            bufs[r % NBUF].at[pl.ds(0, PARTIAL)], lsems[r % NBUF])

    def scat(r):
        return pltpu.make_async_copy(
            bufs[r % NBUF], acc_sh.at[idx_v.at[r]], ssems[r % NBUF])

    def issue_load(r):
        @pl.when(cond_full(r))
        def _():
            load_full(r).start()

        @pl.when(cond_part(r))
        def _():
            load_part(r).start()

    def idx_full(r):
        return pltpu.make_async_copy(
            idx_hbm.at[pl.ds(row_start(r), CHUNK)], idx_v.at[r], isem)

    def idx_part(r):
        return pltpu.make_async_copy(
            idx_hbm.at[pl.ds(row_start(r), PARTIAL)],
            idx_v.at[r].at[pl.ds(0, PARTIAL)], isem)

    for k in range(NBUF - 1):
        issue_load(k)

    for r in range(ROUNDS):
        @pl.when(cond_full(r))
        def _():
            idx_full(r).start()

        @pl.when(cond_part(r))
        def _():
            idx_part(r).start()

    zvec = jnp.zeros((16,), jnp.float32)

    def _zero_row(i, _):
        for j in range(EMB // 16):
            bounce_v[i, pl.ds(j * 16, 16)] = zvec
        return 0

    lax.fori_loop(0, ZROWS, _zero_row, 0)
    pltpu.sync_copy(bounce_v, acc_sh.at[pl.ds(s * ZROWS, ZROWS)])

    for r in range(ROUNDS):
        @pl.when(cond_full(r))
        def _():
            idx_full(r).wait()

        @pl.when(cond_part(r))
        def _():
            idx_part(r).wait()
            pad = jnp.full((16,), BATCH, jnp.int32)
            for j in range(PARTIAL // 16, CHUNK // 16):
                idx_v[r, pl.ds(j * 16, 16)] = pad

    plsc.subcore_barrier()

    for r in range(ROUNDS):
        @pl.when(cond_full(r))
        def _full():
            load_full(r).wait()
            scat(r).start(add=True)

        @pl.when(cond_part(r))
        def _partial():
            load_part(r).wait()
            scat(r).start(add=True)

        nxt = r + NBUF - 1
        if nxt < ROUNDS:
            prev = nxt - NBUF
            if prev >= 0:
                @pl.when(cond_full(prev) | cond_part(prev))
                def _drain():
                    scat(prev).wait()
            issue_load(nxt)

    for r in range(max(0, ROUNDS - NBUF), ROUNDS):
        @pl.when(cond_full(r) | cond_part(r))
        def _drain_tail():
            scat(r).wait()

    plsc.subcore_barrier()

    wrows = BATCH // NS
    pltpu.sync_copy(acc_sh.at[pl.ds(s * wrows, wrows)],
                    bounce_v.at[pl.ds(0, wrows)])
    pltpu.sync_copy(bounce_v.at[pl.ds(0, wrows)],
                    out_hbm.at[c].at[pl.ds(s * wrows, wrows)])


def _head_body(p_ref, w1, b1, g1, bt1, w2, b2, g2, bt2, w3t, b3, o_ref):
    pooled = p_ref[0] + p_ref[1]
    h = jnp.dot(pooled, w1[...], preferred_element_type=jnp.float32) + b1[...]
    h = jnp.maximum(h * (g1[...] * _BN_INV) + bt1[...], 0.0)
    h = jnp.dot(h, w2[...], preferred_element_type=jnp.float32) + b2[...]
    h = jnp.maximum(h * (g2[...] * _BN_INV) + bt2[...], 0.0)
    o_ref[...] = jnp.sum(h * w3t[...], axis=1, keepdims=True) + b3[...]


def _head(partials, W1, b1, g1, bt1, W2, b2, g2, bt2, W3, b3):
    row = lambda v: v.reshape(1, -1)
    return pl.pallas_call(
        _head_body,
        out_shape=jax.ShapeDtypeStruct((BATCH, OUT), jnp.float32),
    )(partials, W1, row(b1), row(g1), row(bt1),
      W2, row(b2), row(g2), row(bt2),
      W3.reshape(1, 2 * EMB), b3.reshape(1, 1))


def kernel(x_0, x_0_batch, num_cells_0, W1, b1, g1, bt1, W2, b2, g2, bt2, W3, b3):
    idx = jnp.squeeze(x_0_batch).astype(jnp.int32)
    partials = _make_sc_segment_sum()(x_0, idx)
    return _head(partials, W1, b1, g1, bt1, W2, b2, g2, bt2, W3, b3)


# speedup vs baseline: 6.8774x; 1.0417x over previous
import functools

import jax
import jax.numpy as jnp
from jax import lax
from jax.experimental import pallas as pl
from jax.experimental.pallas import tpu as pltpu
from jax.experimental.pallas import tpu_sc as plsc

N = 100000
EMB = 128
BATCH = 1024
OUT = 1

NC = 2
NS = 16
NW = NC * NS
CHUNK = 128
NBUF = 6
ROUNDS = 25
PAD_ROWS = NW * ROUNDS * CHUNK
PARTIAL = N % CHUNK
ACC_ROWS = 1152
ZROWS = ACC_ROWS // NS

_BN_INV = 0.9999950000374997


@functools.cache
def _make_sc_segment_sum():
    mesh = plsc.VectorSubcoreMesh(
        core_axis_name="c", subcore_axis_name="s",
        num_cores=NC, num_subcores=NS)
    return pl.kernel(
        _sc_segment_sum_body,
        out_type=jax.ShapeDtypeStruct((NC, BATCH, EMB), jnp.float32),
        mesh=mesh,
        scratch_types=[
            pltpu.VMEM((ROUNDS, CHUNK), jnp.int32),
            [pltpu.VMEM((CHUNK, EMB), jnp.float32) for _ in range(NBUF)],
            pltpu.VMEM((ZROWS, EMB), jnp.float32),
            pltpu.VMEM_SHARED((ACC_ROWS, EMB), jnp.float32),
            [pltpu.SemaphoreType.DMA for _ in range(NBUF)],
            [pltpu.SemaphoreType.DMA for _ in range(NBUF)],
            pltpu.SemaphoreType.DMA,
        ],
    )


def _sc_segment_sum_body(x_hbm, idx_hbm, out_hbm,
                         idx_v, bufs, bounce_v, acc_sh, lsems, ssems, isem):
    c = lax.axis_index("c")
    s = lax.axis_index("s")
    w = c * NS + s

    def row_start(r):
        return (w * ROUNDS + r) * CHUNK

    def cond_full(r):
        return row_start(r) + CHUNK <= N

    def cond_part(r):
        return row_start(r) == N - PARTIAL

    def load_full(r):
        return pltpu.make_async_copy(
            x_hbm.at[pl.ds(row_start(r), CHUNK)], bufs[r % NBUF],
            lsems[r % NBUF])

    def load_part(r):
        return pltpu.make_async_copy(
            x_hbm.at[pl.ds(row_start(r), PARTIAL)],
            bufs[r % NBUF].at[pl.ds(0, PARTIAL)], lsems[r % NBUF])

    def scat(r):
        return pltpu.make_async_copy(
            bufs[r % NBUF], acc_sh.at[idx_v.at[r]], ssems[r % NBUF])

    def issue_load(r):
        @pl.when(cond_full(r))
        def _():
            load_full(r).start()

        @pl.when(cond_part(r))
        def _():
            load_part(r).start()

    def idx_full(r):
        return pltpu.make_async_copy(
            idx_hbm.at[pl.ds(row_start(r), CHUNK)], idx_v.at[r], isem)

    def idx_part(r):
        return pltpu.make_async_copy(
            idx_hbm.at[pl.ds(row_start(r), PARTIAL)],
            idx_v.at[r].at[pl.ds(0, PARTIAL)], isem)

    for k in range(NBUF - 1):
        issue_load(k)

    for r in range(ROUNDS):
        @pl.when(cond_full(r))
        def _():
            idx_full(r).start()

        @pl.when(cond_part(r))
        def _():
            idx_part(r).start()

    zvec = jnp.zeros((16,), jnp.float32)

    def _zero_row(i, _):
        for j in range(EMB // 16):
            bounce_v[i, pl.ds(j * 16, 16)] = zvec
        return 0

    lax.fori_loop(0, ZROWS, _zero_row, 0)
    pltpu.sync_copy(bounce_v, acc_sh.at[pl.ds(s * ZROWS, ZROWS)])

    for r in range(ROUNDS):
        @pl.when(cond_full(r))
        def _():
            idx_full(r).wait()

        @pl.when(cond_part(r))
        def _():
            idx_part(r).wait()
            pad = jnp.full((16,), BATCH, jnp.int32)
            for j in range(PARTIAL // 16, CHUNK // 16):
                idx_v[r, pl.ds(j * 16, 16)] = pad

    plsc.subcore_barrier()

    for r in range(ROUNDS):
        @pl.when(cond_full(r))
        def _full():
            load_full(r).wait()
            scat(r).start(add=True)

        @pl.when(cond_part(r))
        def _partial():
            load_part(r).wait()
            scat(r).start(add=True)

        nxt = r + NBUF - 1
        if nxt < ROUNDS:
            prev = nxt - NBUF
            if prev >= 0:
                @pl.when(cond_full(prev) | cond_part(prev))
                def _drain():
                    scat(prev).wait()
            issue_load(nxt)

    for r in range(max(0, ROUNDS - NBUF), ROUNDS):
        @pl.when(cond_full(r) | cond_part(r))
        def _drain_tail():
            scat(r).wait()

    plsc.subcore_barrier()

    wrows = BATCH // NS
    pltpu.sync_copy(acc_sh.at[pl.ds(s * wrows, wrows)],
                    bounce_v.at[pl.ds(0, wrows)])
    pltpu.sync_copy(bounce_v.at[pl.ds(0, wrows)],
                    out_hbm.at[c].at[pl.ds(s * wrows, wrows)])


def _head_body(p_ref, w1, b1, g1, bt1, w2, b2, g2, bt2, w3t, b3, o_ref):
    pooled = p_ref[0] + p_ref[1]
    h = jnp.dot(pooled, w1[...], preferred_element_type=jnp.float32) + b1[...]
    h = jnp.maximum(h * (g1[...] * _BN_INV) + bt1[...], 0.0)
    h = jnp.dot(h, w2[...], preferred_element_type=jnp.float32) + b2[...]
    h = jnp.maximum(h * (g2[...] * _BN_INV) + bt2[...], 0.0)
    o_ref[...] = jnp.sum(h * w3t[...], axis=1, keepdims=True) + b3[...]


def _head(partials, W1, b1, g1, bt1, W2, b2, g2, bt2, W3, b3):
    row = lambda v: v.reshape(1, -1)
    return pl.pallas_call(
        _head_body,
        out_shape=jax.ShapeDtypeStruct((BATCH, OUT), jnp.float32),
    )(partials, W1, row(b1), row(g1), row(bt1),
      W2, row(b2), row(g2), row(bt2),
      W3.reshape(1, 2 * EMB), b3.reshape(1, 1))


def kernel(x_0, x_0_batch, num_cells_0, W1, b1, g1, bt1, W2, b2, g2, bt2, W3, b3):
    idx = jnp.squeeze(x_0_batch).astype(jnp.int32)
    partials = _make_sc_segment_sum()(x_0, idx)
    return _head(partials, W1, b1, g1, bt1, W2, b2, g2, bt2, W3, b3)
